# baseline (speedup 1.0000x reference)
import functools

import jax
import jax.numpy as jnp
from jax.experimental import pallas as pl
from jax.experimental.pallas import tpu as pltpu

_THRESH = 1000.0
_NO_CODE = -1
_C_TILE = 1024


def _nn_body(xtneg_ref, x2_ref, c_ref, c2_ref, out_ref, min_ref, arg_ref,
             *, c_tile, thresh):
    i = pl.program_id(0)
    dotneg = jax.lax.dot_general(
        c_ref[...], xtneg_ref[...], (((1,), (1,)), ((), ())),
        preferred_element_type=jnp.float32)
    dist = (x2_ref[...] + c2_ref[...]) + dotneg

    lmin = jnp.min(dist, axis=0, keepdims=True)
    larg = jnp.argmin(dist, axis=0).astype(jnp.int32)[None, :] + i * c_tile

    @pl.when(i == 0)
    def _():
        min_ref[...] = lmin
        arg_ref[...] = larg

    @pl.when(i != 0)
    def _():
        better = lmin < min_ref[...]
        arg_ref[...] = jnp.where(better, larg, arg_ref[...])
        min_ref[...] = jnp.minimum(lmin, min_ref[...])

    @pl.when(i == pl.num_programs(0) - 1)
    def _():
        out_ref[...] = jnp.where(min_ref[...] <= thresh, arg_ref[...],
                                 jnp.int32(_NO_CODE))


def kernel(x, training, codes, is_active):
    del training
    b, s, d = x.shape
    n = codes.shape[0]
    r = b * s

    xtneg = -2.0 * x.reshape(r, d)
    x2 = (x ** 2).sum(-1).reshape(1, r)
    c2 = (codes ** 2).sum(-1)
    c2 = jnp.where(is_active, c2, jnp.inf).reshape(n, 1)

    n_steps = n // _C_TILE
    out = pl.pallas_call(
        functools.partial(_nn_body, c_tile=_C_TILE, thresh=_THRESH),
        grid=(n_steps,),
        in_specs=[
            pl.BlockSpec((r, d), lambda i: (0, 0)),
            pl.BlockSpec((1, r), lambda i: (0, 0)),
            pl.BlockSpec((_C_TILE, d), lambda i: (i, 0)),
            pl.BlockSpec((_C_TILE, 1), lambda i: (i, 0)),
        ],
        out_specs=pl.BlockSpec((1, r), lambda i: (0, 0)),
        out_shape=jax.ShapeDtypeStruct((1, r), jnp.int32),
        scratch_shapes=[
            pltpu.VMEM((1, r), jnp.float32),
            pltpu.VMEM((1, r), jnp.int32),
        ],
        compiler_params=pltpu.CompilerParams(
            dimension_semantics=("arbitrary",)),
    )(xtneg, x2, codes, c2)
    return out.reshape(b, s)


# speedup vs baseline: 1.0762x; 1.0762x over previous
import functools

import jax
import jax.numpy as jnp
from jax.experimental import pallas as pl
from jax.experimental.pallas import tpu as pltpu

_THRESH = 1000.0
_NO_CODE = -1
_C_TILE = 2048


def _nn_body(xtneg_ref, x2_ref, c_ref, c2_ref, out_ref, min_ref, arg_ref,
             *, c_tile, thresh):
    i = pl.program_id(0)
    dotneg = jax.lax.dot_general(
        c_ref[...], xtneg_ref[...], (((1,), (0,)), ((), ())),
        preferred_element_type=jnp.float32)
    dist = (x2_ref[...] + c2_ref[...]) + dotneg

    lmin = jnp.min(dist, axis=0, keepdims=True)
    larg = jnp.argmin(dist, axis=0).astype(jnp.int32)[None, :] + i * c_tile

    @pl.when(i == 0)
    def _():
        min_ref[...] = lmin
        arg_ref[...] = larg

    @pl.when(i != 0)
    def _():
        better = lmin < min_ref[...]
        arg_ref[...] = jnp.where(better, larg, arg_ref[...])
        min_ref[...] = jnp.minimum(lmin, min_ref[...])

    @pl.when(i == pl.num_programs(0) - 1)
    def _():
        out_ref[...] = jnp.where(min_ref[...] <= thresh, arg_ref[...],
                                 jnp.int32(_NO_CODE))


def kernel(x, training, codes, is_active):
    del training
    b, s, d = x.shape
    n = codes.shape[0]
    r = b * s

    xtneg = (-2.0 * x.reshape(r, d)).T
    x2 = (x ** 2).sum(-1).reshape(1, r)
    c2 = (codes ** 2).sum(-1)
    c2 = jnp.where(is_active, c2, jnp.inf).reshape(n, 1)

    n_steps = n // _C_TILE
    out = pl.pallas_call(
        functools.partial(_nn_body, c_tile=_C_TILE, thresh=_THRESH),
        grid=(n_steps,),
        in_specs=[
            pl.BlockSpec((d, r), lambda i: (0, 0)),
            pl.BlockSpec((1, r), lambda i: (0, 0)),
            pl.BlockSpec((_C_TILE, d), lambda i: (i, 0)),
            pl.BlockSpec((_C_TILE, 1), lambda i: (i, 0)),
        ],
        out_specs=pl.BlockSpec((1, r), lambda i: (0, 0)),
        out_shape=jax.ShapeDtypeStruct((1, r), jnp.int32),
        scratch_shapes=[
            pltpu.VMEM((1, r), jnp.float32),
            pltpu.VMEM((1, r), jnp.int32),
        ],
        compiler_params=pltpu.CompilerParams(
            dimension_semantics=("arbitrary",)),
    )(xtneg, x2, codes, c2)
    return out.reshape(b, s)


# speedup vs baseline: 1.0852x; 1.0083x over previous
import functools

import jax
import jax.numpy as jnp
from jax.experimental import pallas as pl
from jax.experimental.pallas import tpu as pltpu

_THRESH = 1000.0
_NO_CODE = -1
_C_TILE = 8192


def _nn_body(xtneg_ref, x2_ref, c_ref, c2_ref, out_ref, min_ref, arg_ref,
             *, c_tile, thresh):
    i = pl.program_id(0)
    dotneg = jax.lax.dot_general(
        c_ref[...], xtneg_ref[...], (((1,), (0,)), ((), ())),
        preferred_element_type=jnp.float32)
    dist = (x2_ref[...] + c2_ref[...]) + dotneg

    lmin = jnp.min(dist, axis=0, keepdims=True)
    larg = jnp.argmin(dist, axis=0).astype(jnp.int32)[None, :] + i * c_tile

    @pl.when(i == 0)
    def _():
        min_ref[...] = lmin
        arg_ref[...] = larg

    @pl.when(i != 0)
    def _():
        better = lmin < min_ref[...]
        arg_ref[...] = jnp.where(better, larg, arg_ref[...])
        min_ref[...] = jnp.minimum(lmin, min_ref[...])

    @pl.when(i == pl.num_programs(0) - 1)
    def _():
        out_ref[...] = jnp.where(min_ref[...] <= thresh, arg_ref[...],
                                 jnp.int32(_NO_CODE))


def kernel(x, training, codes, is_active):
    del training
    b, s, d = x.shape
    n = codes.shape[0]
    r = b * s

    xtneg = (-2.0 * x.reshape(r, d)).T
    x2 = (x ** 2).sum(-1).reshape(1, r)
    c2 = (codes ** 2).sum(-1)
    c2 = jnp.where(is_active, c2, jnp.inf).reshape(n, 1)

    n_steps = n // _C_TILE
    out = pl.pallas_call(
        functools.partial(_nn_body, c_tile=_C_TILE, thresh=_THRESH),
        grid=(n_steps,),
        in_specs=[
            pl.BlockSpec((d, r), lambda i: (0, 0)),
            pl.BlockSpec((1, r), lambda i: (0, 0)),
            pl.BlockSpec((_C_TILE, d), lambda i: (i, 0)),
            pl.BlockSpec((_C_TILE, 1), lambda i: (i, 0)),
        ],
        out_specs=pl.BlockSpec((1, r), lambda i: (0, 0)),
        out_shape=jax.ShapeDtypeStruct((1, r), jnp.int32),
        scratch_shapes=[
            pltpu.VMEM((1, r), jnp.float32),
            pltpu.VMEM((1, r), jnp.int32),
        ],
        compiler_params=pltpu.CompilerParams(
            dimension_semantics=("arbitrary",)),
    )(xtneg, x2, codes, c2)
    return out.reshape(b, s)


# speedup vs baseline: 1.1039x; 1.0172x over previous
import functools

import jax
import jax.numpy as jnp
from jax.experimental import pallas as pl
from jax.experimental.pallas import tpu as pltpu

_THRESH = 1000.0
_NO_CODE = -1
_C_TILE = 4096


def _nn_body(xtneg_ref, x2_ref, c_ref, c2_ref, out_ref, min_ref, arg_ref,
             *, c_tile, thresh):
    i = pl.program_id(0)
    dotneg = jax.lax.dot_general(
        c_ref[...], xtneg_ref[...], (((1,), (0,)), ((), ())),
        preferred_element_type=jnp.float32)
    dist = (x2_ref[...] + c2_ref[...]) + dotneg

    lmin = jnp.min(dist, axis=0, keepdims=True)
    larg = jnp.argmin(dist, axis=0).astype(jnp.int32)[None, :] + i * c_tile

    @pl.when(i == 0)
    def _():
        min_ref[...] = lmin
        arg_ref[...] = larg

    @pl.when(i != 0)
    def _():
        better = lmin < min_ref[...]
        arg_ref[...] = jnp.where(better, larg, arg_ref[...])
        min_ref[...] = jnp.minimum(lmin, min_ref[...])

    @pl.when(i == pl.num_programs(0) - 1)
    def _():
        out_ref[...] = jnp.where(min_ref[...] <= thresh, arg_ref[...],
                                 jnp.int32(_NO_CODE))


def kernel(x, training, codes, is_active):
    del training
    b, s, d = x.shape
    n = codes.shape[0]
    r = b * s

    xtneg = (-2.0 * x.reshape(r, d)).T
    x2 = (x ** 2).sum(-1).reshape(1, r)
    c2 = ((codes ** 2).sum(-1) / is_active.astype(jnp.float32)).reshape(n, 1)

    n_steps = n // _C_TILE
    out = pl.pallas_call(
        functools.partial(_nn_body, c_tile=_C_TILE, thresh=_THRESH),
        grid=(n_steps,),
        in_specs=[
            pl.BlockSpec((d, r), lambda i: (0, 0)),
            pl.BlockSpec((1, r), lambda i: (0, 0)),
            pl.BlockSpec((_C_TILE, d), lambda i: (i, 0)),
            pl.BlockSpec((_C_TILE, 1), lambda i: (i, 0)),
        ],
        out_specs=pl.BlockSpec((1, r), lambda i: (0, 0)),
        out_shape=jax.ShapeDtypeStruct((1, r), jnp.int32),
        scratch_shapes=[
            pltpu.VMEM((1, r), jnp.float32),
            pltpu.VMEM((1, r), jnp.int32),
        ],
        compiler_params=pltpu.CompilerParams(
            dimension_semantics=("arbitrary",)),
    )(xtneg, x2, codes, c2)
    return out.reshape(b, s)


# speedup vs baseline: 1.2253x; 1.1100x over previous
import functools

import jax
import jax.numpy as jnp
from jax.experimental import pallas as pl
from jax.experimental.pallas import tpu as pltpu

_THRESH = 1000.0
_NO_CODE = -1
_C_TILE = 4096


def _nn_body(xtneg_ref, x2_ref, c_ref, c2_ref, out_ref, min_ref, arg_ref,
             *, c_tile, thresh):
    i = pl.program_id(0)
    dotneg = jax.lax.dot_general(
        c_ref[...], xtneg_ref[...], (((1,), (0,)), ((), ())),
        preferred_element_type=jnp.float32)
    dist = (x2_ref[...] + c2_ref[...]) + dotneg

    lmin = jnp.min(dist, axis=0, keepdims=True)
    larg = jnp.argmin(dist, axis=0).astype(jnp.int32)[None, :] + i * c_tile

    @pl.when(i == 0)
    def _():
        min_ref[...] = lmin
        arg_ref[...] = larg

    @pl.when(i != 0)
    def _():
        better = lmin < min_ref[...]
        arg_ref[...] = jnp.where(better, larg, arg_ref[...])
        min_ref[...] = jnp.minimum(lmin, min_ref[...])

    @pl.when(i == pl.num_programs(0) - 1)
    def _():
        out_ref[...] = jnp.where(min_ref[...] <= thresh, arg_ref[...],
                                 jnp.int32(_NO_CODE))


def kernel(x, training, codes, is_active):
    del training
    b, s, d = x.shape
    n = codes.shape[0]
    r = b * s

    xtneg = (-2.0 * x.reshape(r, d)).T
    x2 = (x ** 2).sum(-1).reshape(1, r)
    c2 = (codes ** 2).sum(-1)
    c2 = jnp.where(is_active, c2, jnp.inf).reshape(n, 1)

    n_steps = n // _C_TILE
    out = pl.pallas_call(
        functools.partial(_nn_body, c_tile=_C_TILE, thresh=_THRESH),
        grid=(n_steps,),
        in_specs=[
            pl.BlockSpec((d, r), lambda i: (0, 0)),
            pl.BlockSpec((1, r), lambda i: (0, 0)),
            pl.BlockSpec((_C_TILE, d), lambda i: (i, 0)),
            pl.BlockSpec((_C_TILE, 1), lambda i: (i, 0)),
        ],
        out_specs=pl.BlockSpec((1, r), lambda i: (0, 0)),
        out_shape=jax.ShapeDtypeStruct((1, r), jnp.int32),
        scratch_shapes=[
            pltpu.VMEM((1, r), jnp.float32),
            pltpu.VMEM((1, r), jnp.int32),
        ],
        compiler_params=pltpu.CompilerParams(
            dimension_semantics=("arbitrary",),
            allow_input_fusion=[True, False, False, False]),
    )(xtneg, x2, codes, c2)
    return out.reshape(b, s)


# speedup vs baseline: 1.2282x; 1.0024x over previous
import functools

import jax
import jax.numpy as jnp
from jax.experimental import pallas as pl
from jax.experimental.pallas import tpu as pltpu

_THRESH = 1000.0
_NO_CODE = -1
_C_TILE = 4096


def _nn_body(xtneg_ref, x2_ref, c_ref, c2_ref, out_ref, min_ref, arg_ref,
             *, c_tile, thresh):
    i = pl.program_id(0)
    dotneg = jax.lax.dot_general(
        c_ref[...], xtneg_ref[...], (((1,), (0,)), ((), ())),
        preferred_element_type=jnp.float32)
    dist = (x2_ref[...] + c2_ref[...]) + dotneg

    lmin = jnp.min(dist, axis=0, keepdims=True)
    larg = jnp.argmin(dist, axis=0).astype(jnp.int32)[None, :] + i * c_tile

    @pl.when(i == 0)
    def _():
        min_ref[...] = lmin
        arg_ref[...] = larg

    @pl.when(i != 0)
    def _():
        better = lmin < min_ref[...]
        arg_ref[...] = jnp.where(better, larg, arg_ref[...])
        min_ref[...] = jnp.minimum(lmin, min_ref[...])

    @pl.when(i == pl.num_programs(0) - 1)
    def _():
        out_ref[...] = jnp.where(min_ref[...] <= thresh, arg_ref[...],
                                 jnp.int32(_NO_CODE))


def kernel(x, training, codes, is_active):
    del training
    b, s, d = x.shape
    n = codes.shape[0]
    r = b * s

    xtneg = (-2.0 * x.reshape(r, d)).T
    x2 = (x ** 2).sum(-1).reshape(1, r)
    c2 = (codes ** 2).sum(-1)
    c2 = jnp.where(is_active, c2, jnp.inf).reshape(n, 1)

    n_steps = n // _C_TILE
    out = pl.pallas_call(
        functools.partial(_nn_body, c_tile=_C_TILE, thresh=_THRESH),
        grid=(n_steps,),
        in_specs=[
            pl.BlockSpec((d, r), lambda i: (0, 0)),
            pl.BlockSpec((1, r), lambda i: (0, 0)),
            pl.BlockSpec((_C_TILE, d), lambda i: (i, 0)),
            pl.BlockSpec((_C_TILE, 1), lambda i: (i, 0)),
        ],
        out_specs=pl.BlockSpec((1, r), lambda i: (0, 0)),
        out_shape=jax.ShapeDtypeStruct((1, r), jnp.int32),
        scratch_shapes=[
            pltpu.VMEM((1, r), jnp.float32),
            pltpu.VMEM((1, r), jnp.int32),
        ],
        compiler_params=pltpu.CompilerParams(
            dimension_semantics=("arbitrary",),
            allow_input_fusion=[True, True, True, True]),
    )(xtneg, x2, codes, c2)
    return out.reshape(b, s)
